# baseline (speedup 1.0000x reference)
import functools

import jax
import jax.numpy as jnp
from jax import lax
from jax.experimental import pallas as pl
from jax.experimental.pallas import tpu as pltpu
from jax.experimental.pallas import tpu_sc as plsc

_VOCAB = 1000000
_EMB = 32
_BATCH = 16384
_FIELDS = 26
_N = _BATCH * _FIELDS
_NW = 32
_B_PER_W = _N // _NW
_CHUNK = 1024
_NCHUNKS = _B_PER_W // _CHUNK
_NB = 3

_WSTRIDE = 250112
_TBLK = _WSTRIDE // 128
_IN_BLKS = (_VOCAB + 127) // 128 - 1


def _transpose_body(x0_ref, x1_ref, x2_ref, x3_ref, o_ref):
    o_ref[...] = jnp.concatenate(
        [x0_ref[...].T, x1_ref[...].T, x2_ref[...].T, x3_ref[...].T], axis=1
    )


def _in_spec(q):
    return pl.BlockSpec(
        (32, 128), lambda j, q=q: (0, jnp.minimum(j + _TBLK * q, _IN_BLKS))
    )


_pack_table = pl.pallas_call(
    _transpose_body,
    grid=(_TBLK,),
    in_specs=[_in_spec(0), _in_spec(1), _in_spec(2), _in_spec(3)],
    out_specs=pl.BlockSpec((128, 128), lambda j: (j, 0)),
    out_shape=jax.ShapeDtypeStruct((_WSTRIDE, 128), jnp.float32),
)

_mesh = plsc.VectorSubcoreMesh(core_axis_name="c", subcore_axis_name="s")


@functools.partial(
    pl.kernel,
    mesh=_mesh,
    out_type=jax.ShapeDtypeStruct((_N, _EMB), jnp.float32),
    scratch_types=[
        pltpu.VMEM((_B_PER_W,), jnp.int32),
        pltpu.VMEM((_NB, _CHUNK, _EMB), jnp.float32),
        pltpu.SemaphoreType.DMA((_NB,)),
        pltpu.SemaphoreType.DMA((_NB,)),
    ],
    compiler_params=pltpu.CompilerParams(use_tc_tiling_on_sc=False),
)
def _emb_lookup(idx_hbm, table_hbm, out_hbm, idx_v, rows_v, gsems, wsems):
    wid = lax.axis_index("s") * 2 + lax.axis_index("c")
    base = wid * _B_PER_W
    pltpu.sync_copy(idx_hbm.at[pl.ds(base, _B_PER_W)], idx_v)

    def gather(j):
        return pltpu.async_copy(
            table_hbm.at[idx_v.at[pl.ds(j * _CHUNK, _CHUNK)]],
            rows_v.at[j % _NB],
            gsems.at[j % _NB],
        )

    def write(j):
        return pltpu.async_copy(
            rows_v.at[j % _NB],
            out_hbm.at[pl.ds(base + j * _CHUNK, _CHUNK)],
            wsems.at[j % _NB],
        )

    gathers, writes = {}, {}
    waited = set()
    for j in range(min(2, _NCHUNKS)):
        gathers[j] = gather(j)
    for i in range(_NCHUNKS):
        gathers[i].wait()
        writes[i] = write(i)
        j = i + 2
        if j < _NCHUNKS:
            if j - _NB >= 0:
                writes[j - _NB].wait()
                waited.add(j - _NB)
            gathers[j] = gather(j)
    for i in range(_NCHUNKS):
        if i not in waited:
            writes[i].wait()


def kernel(x, weight):
    wt = weight.T
    w128 = _pack_table(wt, wt, wt, wt)
    wlin = w128.reshape(_WSTRIDE * 4, _EMB)
    v = x.reshape(_N)
    q = v // _WSTRIDE
    rows = 4 * v - (4 * _WSTRIDE - 1) * q
    out = _emb_lookup(rows, wlin)
    return out.reshape(_BATCH, _FIELDS, _EMB)


# speedup vs baseline: 2.5045x; 2.5045x over previous
import functools

import jax
import jax.numpy as jnp
from jax import lax
from jax.experimental import pallas as pl
from jax.experimental.pallas import tpu as pltpu
from jax.experimental.pallas import tpu_sc as plsc

_VOCAB = 1000000
_EMB = 32
_BATCH = 16384
_FIELDS = 26
_N = _BATCH * _FIELDS
_NW = 32
_B_PER_W = _N // _NW
_CHUNK = 1024
_NCHUNKS = _B_PER_W // _CHUNK
_NB = 3

_TCOLS = 2048
_TBLK = 123
_WSTRIDE = _TCOLS * _TBLK
_IN_BLKS = (_VOCAB + _TCOLS - 1) // _TCOLS - 1


def _transpose_body(x0_ref, x1_ref, x2_ref, x3_ref, o_ref):
    o_ref[...] = jnp.concatenate(
        [x0_ref[...].T, x1_ref[...].T, x2_ref[...].T, x3_ref[...].T], axis=1
    )


def _in_spec(q):
    return pl.BlockSpec(
        (32, _TCOLS), lambda j, q=q: (0, jnp.minimum(j + _TBLK * q, _IN_BLKS))
    )


_pack_table = pl.pallas_call(
    _transpose_body,
    grid=(_TBLK,),
    in_specs=[_in_spec(0), _in_spec(1), _in_spec(2), _in_spec(3)],
    out_specs=pl.BlockSpec((_TCOLS, 128), lambda j: (j, 0)),
    out_shape=jax.ShapeDtypeStruct((_WSTRIDE, 128), jnp.float32),
)

_mesh = plsc.VectorSubcoreMesh(core_axis_name="c", subcore_axis_name="s")


@functools.partial(
    pl.kernel,
    mesh=_mesh,
    out_type=jax.ShapeDtypeStruct((_N, _EMB), jnp.float32),
    scratch_types=[
        pltpu.VMEM((_B_PER_W,), jnp.int32),
        pltpu.VMEM((_NB, _CHUNK, _EMB), jnp.float32),
        pltpu.SemaphoreType.DMA((_NB,)),
        pltpu.SemaphoreType.DMA((_NB,)),
    ],
    compiler_params=pltpu.CompilerParams(use_tc_tiling_on_sc=False),
)
def _emb_lookup(idx_hbm, table_hbm, out_hbm, idx_v, rows_v, gsems, wsems):
    wid = lax.axis_index("s") * 2 + lax.axis_index("c")
    base = wid * _B_PER_W
    pltpu.sync_copy(idx_hbm.at[pl.ds(base, _B_PER_W)], idx_v)

    def gather(j):
        return pltpu.async_copy(
            table_hbm.at[idx_v.at[pl.ds(j * _CHUNK, _CHUNK)]],
            rows_v.at[j % _NB],
            gsems.at[j % _NB],
        )

    def write(j):
        return pltpu.async_copy(
            rows_v.at[j % _NB],
            out_hbm.at[pl.ds(base + j * _CHUNK, _CHUNK)],
            wsems.at[j % _NB],
        )

    gathers, writes = {}, {}
    waited = set()
    for j in range(min(2, _NCHUNKS)):
        gathers[j] = gather(j)
    for i in range(_NCHUNKS):
        gathers[i].wait()
        writes[i] = write(i)
        j = i + 2
        if j < _NCHUNKS:
            if j - _NB >= 0:
                writes[j - _NB].wait()
                waited.add(j - _NB)
            gathers[j] = gather(j)
    for i in range(_NCHUNKS):
        if i not in waited:
            writes[i].wait()


def kernel(x, weight):
    wt = weight.T
    w128 = _pack_table(wt, wt, wt, wt)
    wlin = w128.reshape(_WSTRIDE * 4, _EMB)
    v = x.reshape(_N)
    q = v // _WSTRIDE
    rows = 4 * v - (4 * _WSTRIDE - 1) * q
    out = _emb_lookup(rows, wlin)
    return out.reshape(_BATCH, _FIELDS, _EMB)


# speedup vs baseline: 2.6391x; 1.0537x over previous
import functools

import jax
import jax.numpy as jnp
from jax import lax
from jax.experimental import pallas as pl
from jax.experimental.pallas import tpu as pltpu
from jax.experimental.pallas import tpu_sc as plsc

_VOCAB = 1000000
_EMB = 32
_BATCH = 16384
_FIELDS = 26
_N = _BATCH * _FIELDS
_NW = 32
_B_PER_W = _N // _NW
_CHUNK = 1024
_NCHUNKS = _B_PER_W // _CHUNK
_NB = 3

_TCOLS = 2048
_TBLK = 123
_WSTRIDE = _TCOLS * _TBLK
_IN_BLKS = (_VOCAB + _TCOLS - 1) // _TCOLS - 1


def _transpose_body(x0_ref, x1_ref, x2_ref, x3_ref, o_ref):
    eye = jnp.eye(32, dtype=jnp.float32)
    dn = (((0,), (0,)), ((), ()))

    def t(x_ref):
        return jax.lax.dot_general(
            x_ref[...], eye, dn, preferred_element_type=jnp.float32
        )

    o_ref[...] = jnp.concatenate(
        [t(x0_ref), t(x1_ref), t(x2_ref), t(x3_ref)], axis=1
    )


def _in_spec(q):
    return pl.BlockSpec(
        (32, _TCOLS), lambda j, q=q: (0, jnp.minimum(j + _TBLK * q, _IN_BLKS))
    )


_pack_table = pl.pallas_call(
    _transpose_body,
    grid=(_TBLK,),
    in_specs=[_in_spec(0), _in_spec(1), _in_spec(2), _in_spec(3)],
    out_specs=pl.BlockSpec((_TCOLS, 128), lambda j: (j, 0)),
    out_shape=jax.ShapeDtypeStruct((_WSTRIDE, 128), jnp.float32),
)

_mesh = plsc.VectorSubcoreMesh(core_axis_name="c", subcore_axis_name="s")


@functools.partial(
    pl.kernel,
    mesh=_mesh,
    out_type=jax.ShapeDtypeStruct((_N, _EMB), jnp.float32),
    scratch_types=[
        pltpu.VMEM((_B_PER_W,), jnp.int32),
        pltpu.VMEM((_NB, _CHUNK, _EMB), jnp.float32),
        pltpu.SemaphoreType.DMA((_NB,)),
        pltpu.SemaphoreType.DMA((_NB,)),
    ],
    compiler_params=pltpu.CompilerParams(use_tc_tiling_on_sc=False),
)
def _emb_lookup(idx_hbm, table_hbm, out_hbm, idx_v, rows_v, gsems, wsems):
    wid = lax.axis_index("s") * 2 + lax.axis_index("c")
    base = wid * _B_PER_W
    pltpu.sync_copy(idx_hbm.at[pl.ds(base, _B_PER_W)], idx_v)

    def gather(j):
        return pltpu.async_copy(
            table_hbm.at[idx_v.at[pl.ds(j * _CHUNK, _CHUNK)]],
            rows_v.at[j % _NB],
            gsems.at[j % _NB],
        )

    def write(j):
        return pltpu.async_copy(
            rows_v.at[j % _NB],
            out_hbm.at[pl.ds(base + j * _CHUNK, _CHUNK)],
            wsems.at[j % _NB],
        )

    gathers, writes = {}, {}
    waited = set()
    for j in range(min(2, _NCHUNKS)):
        gathers[j] = gather(j)
    for i in range(_NCHUNKS):
        gathers[i].wait()
        writes[i] = write(i)
        j = i + 2
        if j < _NCHUNKS:
            if j - _NB >= 0:
                writes[j - _NB].wait()
                waited.add(j - _NB)
            gathers[j] = gather(j)
    for i in range(_NCHUNKS):
        if i not in waited:
            writes[i].wait()


def kernel(x, weight):
    wt = weight.T
    w128 = _pack_table(wt, wt, wt, wt)
    wlin = w128.reshape(_WSTRIDE * 4, _EMB)
    q = x // _WSTRIDE
    rows = (4 * x - (4 * _WSTRIDE - 1) * q).reshape(_N)
    out = _emb_lookup(rows, wlin)
    return out.reshape(_BATCH, _FIELDS, _EMB)


# speedup vs baseline: 2.6419x; 1.0011x over previous
import functools

import jax
import jax.numpy as jnp
from jax import lax
from jax.experimental import pallas as pl
from jax.experimental.pallas import tpu as pltpu
from jax.experimental.pallas import tpu_sc as plsc

_VOCAB = 1000000
_EMB = 32
_BATCH = 16384
_FIELDS = 26
_N = _BATCH * _FIELDS
_NW = 32
_B_PER_W = _N // _NW
_BW = _BATCH // _NW
_CHUNK = 1024
_NCHUNKS = _B_PER_W // _CHUNK
_NB = 3

_TCOLS = 2048
_TBLK = 123
_WSTRIDE = _TCOLS * _TBLK
_IN_BLKS = (_VOCAB + _TCOLS - 1) // _TCOLS - 1


def _transpose_body(x0_ref, x1_ref, x2_ref, x3_ref, o_ref):
    eye = jnp.eye(32, dtype=jnp.float32)
    dn = (((0,), (0,)), ((), ()))

    def t(x_ref):
        return jax.lax.dot_general(
            x_ref[...], eye, dn, preferred_element_type=jnp.float32
        )

    o_ref[...] = jnp.concatenate(
        [t(x0_ref), t(x1_ref), t(x2_ref), t(x3_ref)], axis=1
    )


def _in_spec(q):
    return pl.BlockSpec(
        (32, _TCOLS), lambda j, q=q: (0, jnp.minimum(j + _TBLK * q, _IN_BLKS))
    )


_pack_table = pl.pallas_call(
    _transpose_body,
    grid=(_TBLK,),
    in_specs=[_in_spec(0), _in_spec(1), _in_spec(2), _in_spec(3)],
    out_specs=pl.BlockSpec((_TCOLS, 128), lambda j: (j, 0)),
    out_shape=jax.ShapeDtypeStruct((_WSTRIDE, 128), jnp.float32),
    compiler_params=pltpu.CompilerParams(fuse_transposed_lhs_in_matmul=True),
)

_mesh = plsc.VectorSubcoreMesh(core_axis_name="c", subcore_axis_name="s")


@functools.partial(
    pl.kernel,
    mesh=_mesh,
    out_type=jax.ShapeDtypeStruct((_N, _EMB), jnp.float32),
    scratch_types=[
        pltpu.VMEM((_B_PER_W,), jnp.int32),
        pltpu.VMEM((_NB, _CHUNK, _EMB), jnp.float32),
        pltpu.SemaphoreType.DMA((_NB,)),
        pltpu.SemaphoreType.DMA((_NB,)),
    ],
    compiler_params=pltpu.CompilerParams(use_tc_tiling_on_sc=False),
)
def _emb_lookup(idx_hbm, table_hbm, out_hbm, idx_v, rows_v, gsems, wsems):
    wid = lax.axis_index("s") * 2 + lax.axis_index("c")
    base = wid * _B_PER_W
    bbase = wid * _BW
    pltpu.sync_copy(idx_hbm.at[pl.ds(base, _B_PER_W)], idx_v)

    def gather(j):
        return pltpu.async_copy(
            table_hbm.at[idx_v.at[pl.ds(j * _CHUNK, _CHUNK)]],
            rows_v.at[j % _NB],
            gsems.at[j % _NB],
        )

    def write(j):
        return [
            pltpu.async_copy(
                rows_v.at[j % _NB],
                out_hbm.at[pl.ds(base + j * _CHUNK, _CHUNK)],
                wsems.at[j % _NB],
            )
        ]

    gathers, writes = {}, {}
    waited = set()
    for j in range(min(2, _NCHUNKS)):
        gathers[j] = gather(j)
    for i in range(_NCHUNKS):
        gathers[i].wait()
        writes[i] = write(i)
        j = i + 2
        if j < _NCHUNKS:
            if j - _NB >= 0:
                for h in writes[j - _NB]:
                    h.wait()
                waited.add(j - _NB)
            gathers[j] = gather(j)
    for i in range(_NCHUNKS):
        if i not in waited:
            for h in writes[i]:
                h.wait()


def kernel(x, weight):
    wt = weight.T
    w128 = _pack_table(wt, wt, wt, wt)
    wlin = w128.reshape(_WSTRIDE * 4, _EMB)
    q = x // _WSTRIDE
    rows = (4 * x - (4 * _WSTRIDE - 1) * q).reshape(_N)
    out = _emb_lookup(rows, wlin)
    return out.reshape(_BATCH, _FIELDS, _EMB)


# speedup vs baseline: 2.6423x; 1.0002x over previous
import functools

import jax
import jax.numpy as jnp
from jax import lax
from jax.experimental import pallas as pl
from jax.experimental.pallas import tpu as pltpu
from jax.experimental.pallas import tpu_sc as plsc

_VOCAB = 1000000
_EMB = 32
_BATCH = 16384
_FIELDS = 26
_N = _BATCH * _FIELDS
_NW = 32
_B_PER_W = _N // _NW
_BW = _BATCH // _NW
_CHUNK = 1024
_NCHUNKS = _B_PER_W // _CHUNK
_NB = 3

_TCOLS = 2048
_TBLK = 123
_WSTRIDE = _TCOLS * _TBLK
_IN_BLKS = (_VOCAB + _TCOLS - 1) // _TCOLS - 1


def _transpose_body(x0_ref, x1_ref, x2_ref, x3_ref, o_ref):
    o_ref[...] = jnp.concatenate(
        [x0_ref[...].T, x1_ref[...].T, x2_ref[...].T, x3_ref[...].T], axis=1
    )


def _in_spec(q):
    return pl.BlockSpec(
        (32, _TCOLS), lambda j, q=q: (0, jnp.minimum(j + _TBLK * q, _IN_BLKS))
    )


_pack_table = pl.pallas_call(
    _transpose_body,
    grid=(_TBLK,),
    in_specs=[_in_spec(0), _in_spec(1), _in_spec(2), _in_spec(3)],
    out_specs=pl.BlockSpec((_TCOLS, 128), lambda j: (j, 0)),
    out_shape=jax.ShapeDtypeStruct((_WSTRIDE, 128), jnp.float32),
)

_mesh = plsc.VectorSubcoreMesh(core_axis_name="c", subcore_axis_name="s")


@functools.partial(
    pl.kernel,
    mesh=_mesh,
    out_type=jax.ShapeDtypeStruct((_N, _EMB), jnp.float32),
    scratch_types=[
        pltpu.VMEM((_B_PER_W,), jnp.int32),
        pltpu.VMEM((_NB, _CHUNK, _EMB), jnp.float32),
        pltpu.SemaphoreType.DMA((_NB,)),
        pltpu.SemaphoreType.DMA((_NB,)),
    ],
    compiler_params=pltpu.CompilerParams(use_tc_tiling_on_sc=False),
)
def _emb_lookup(idx_hbm, table_hbm, out_hbm, idx_v, rows_v, gsems, wsems):
    wid = lax.axis_index("s") * 2 + lax.axis_index("c")
    base = wid * _B_PER_W
    pltpu.sync_copy(idx_hbm.at[pl.ds(base, _B_PER_W)], idx_v)

    def gather(j):
        return pltpu.async_copy(
            table_hbm.at[idx_v.at[pl.ds(j * _CHUNK, _CHUNK)]],
            rows_v.at[j % _NB],
            gsems.at[j % _NB],
        )

    def write(j):
        return [
            pltpu.async_copy(
                rows_v.at[j % _NB],
                out_hbm.at[pl.ds(base + j * _CHUNK, _CHUNK)],
                wsems.at[j % _NB],
            )
        ]

    gathers, writes = {}, {}
    waited = set()
    for j in range(min(2, _NCHUNKS)):
        gathers[j] = gather(j)
    for i in range(_NCHUNKS):
        gathers[i].wait()
        writes[i] = write(i)
        j = i + 2
        if j < _NCHUNKS:
            if j - _NB >= 0:
                for h in writes[j - _NB]:
                    h.wait()
                waited.add(j - _NB)
            gathers[j] = gather(j)
    for i in range(_NCHUNKS):
        if i not in waited:
            for h in writes[i]:
                h.wait()


def kernel(x, weight):
    wt = weight.T
    w128 = _pack_table(wt, wt, wt, wt)
    wlin = w128.reshape(_WSTRIDE * 4, _EMB)
    q = x // _WSTRIDE
    rows = (4 * x - (4 * _WSTRIDE - 1) * q).reshape(_N)
    out = _emb_lookup(rows, wlin)
    return out.reshape(_BATCH, _FIELDS, _EMB)


# speedup vs baseline: 2.6768x; 1.0130x over previous
import functools

import jax
import jax.numpy as jnp
from jax import lax
from jax.experimental import pallas as pl
from jax.experimental.pallas import tpu as pltpu
from jax.experimental.pallas import tpu_sc as plsc

_VOCAB = 1000000
_EMB = 32
_BATCH = 16384
_FIELDS = 26
_N = _BATCH * _FIELDS
_NW = 32
_B_PER_W = _N // _NW
_BW = _BATCH // _NW
_CHUNK = 1024
_NCHUNKS = _B_PER_W // _CHUNK
_NB = 3

_TCOLS = 4096
_TBLK = 62
_WSTRIDE = _TCOLS * _TBLK
_IN_BLKS = (_VOCAB + _TCOLS - 1) // _TCOLS - 1


def _transpose_body(x0_ref, x1_ref, x2_ref, x3_ref, o_ref):
    o_ref[...] = jnp.concatenate(
        [x0_ref[...].T, x1_ref[...].T, x2_ref[...].T, x3_ref[...].T], axis=1
    )


def _in_spec(q):
    return pl.BlockSpec(
        (32, _TCOLS), lambda j, q=q: (0, jnp.minimum(j + _TBLK * q, _IN_BLKS))
    )


_pack_table = pl.pallas_call(
    _transpose_body,
    grid=(_TBLK,),
    in_specs=[_in_spec(0), _in_spec(1), _in_spec(2), _in_spec(3)],
    out_specs=pl.BlockSpec((_TCOLS, 128), lambda j: (j, 0)),
    out_shape=jax.ShapeDtypeStruct((_WSTRIDE, 128), jnp.float32),
)

_mesh = plsc.VectorSubcoreMesh(core_axis_name="c", subcore_axis_name="s")


@functools.partial(
    pl.kernel,
    mesh=_mesh,
    out_type=jax.ShapeDtypeStruct((_N, _EMB), jnp.float32),
    scratch_types=[
        pltpu.VMEM((_B_PER_W,), jnp.int32),
        pltpu.VMEM((_NB, _CHUNK, _EMB), jnp.float32),
        pltpu.SemaphoreType.DMA((_NB,)),
        pltpu.SemaphoreType.DMA((_NB,)),
    ],
    compiler_params=pltpu.CompilerParams(use_tc_tiling_on_sc=False),
)
def _emb_lookup(idx_hbm, table_hbm, out_hbm, idx_v, rows_v, gsems, wsems):
    wid = lax.axis_index("s") * 2 + lax.axis_index("c")
    base = wid * _B_PER_W
    pltpu.sync_copy(idx_hbm.at[pl.ds(base, _B_PER_W)], idx_v)

    def gather(j):
        return pltpu.async_copy(
            table_hbm.at[idx_v.at[pl.ds(j * _CHUNK, _CHUNK)]],
            rows_v.at[j % _NB],
            gsems.at[j % _NB],
        )

    def write(j):
        return [
            pltpu.async_copy(
                rows_v.at[j % _NB],
                out_hbm.at[pl.ds(base + j * _CHUNK, _CHUNK)],
                wsems.at[j % _NB],
            )
        ]

    gathers, writes = {}, {}
    waited = set()
    for j in range(min(2, _NCHUNKS)):
        gathers[j] = gather(j)
    for i in range(_NCHUNKS):
        gathers[i].wait()
        writes[i] = write(i)
        j = i + 2
        if j < _NCHUNKS:
            if j - _NB >= 0:
                for h in writes[j - _NB]:
                    h.wait()
                waited.add(j - _NB)
            gathers[j] = gather(j)
    for i in range(_NCHUNKS):
        if i not in waited:
            for h in writes[i]:
                h.wait()


def kernel(x, weight):
    wt = weight.T
    w128 = _pack_table(wt, wt, wt, wt)
    wlin = w128.reshape(_WSTRIDE * 4, _EMB)
    q = x // _WSTRIDE
    rows = (4 * x - (4 * _WSTRIDE - 1) * q).reshape(_N)
    out = _emb_lookup(rows, wlin)
    return out.reshape(_BATCH, _FIELDS, _EMB)


# speedup vs baseline: 2.6971x; 1.0076x over previous
import functools

import jax
import jax.numpy as jnp
from jax import lax
from jax.experimental import pallas as pl
from jax.experimental.pallas import tpu as pltpu
from jax.experimental.pallas import tpu_sc as plsc

_VOCAB = 1000000
_EMB = 32
_BATCH = 16384
_FIELDS = 26
_N = _BATCH * _FIELDS
_NW = 32
_B_PER_W = _N // _NW
_BW = _BATCH // _NW
_CHUNK = 1024
_NCHUNKS = _B_PER_W // _CHUNK
_NB = 3

_TCOLS = 8192
_TBLK = 31
_WSTRIDE = _TCOLS * _TBLK
_IN_BLKS = (_VOCAB + _TCOLS - 1) // _TCOLS - 1


def _transpose_body(x0_ref, x1_ref, x2_ref, x3_ref, o_ref):
    o_ref[...] = jnp.concatenate(
        [x0_ref[...].T, x1_ref[...].T, x2_ref[...].T, x3_ref[...].T], axis=1
    )


def _in_spec(q):
    return pl.BlockSpec(
        (32, _TCOLS), lambda j, q=q: (0, jnp.minimum(j + _TBLK * q, _IN_BLKS))
    )


_pack_table = pl.pallas_call(
    _transpose_body,
    grid=(_TBLK,),
    in_specs=[_in_spec(0), _in_spec(1), _in_spec(2), _in_spec(3)],
    out_specs=pl.BlockSpec((_TCOLS, 128), lambda j: (j, 0)),
    out_shape=jax.ShapeDtypeStruct((_WSTRIDE, 128), jnp.float32),
)

_mesh = plsc.VectorSubcoreMesh(core_axis_name="c", subcore_axis_name="s")


@functools.partial(
    pl.kernel,
    mesh=_mesh,
    out_type=jax.ShapeDtypeStruct((_N, _EMB), jnp.float32),
    scratch_types=[
        pltpu.VMEM((_B_PER_W,), jnp.int32),
        pltpu.VMEM((_NB, _CHUNK, _EMB), jnp.float32),
        pltpu.SemaphoreType.DMA((_NB,)),
        pltpu.SemaphoreType.DMA((_NB,)),
    ],
    compiler_params=pltpu.CompilerParams(use_tc_tiling_on_sc=False),
)
def _emb_lookup(idx_hbm, table_hbm, out_hbm, idx_v, rows_v, gsems, wsems):
    wid = lax.axis_index("s") * 2 + lax.axis_index("c")
    base = wid * _B_PER_W
    pltpu.sync_copy(idx_hbm.at[pl.ds(base, _B_PER_W)], idx_v)

    def gather(j):
        return pltpu.async_copy(
            table_hbm.at[idx_v.at[pl.ds(j * _CHUNK, _CHUNK)]],
            rows_v.at[j % _NB],
            gsems.at[j % _NB],
        )

    def write(j):
        return [
            pltpu.async_copy(
                rows_v.at[j % _NB],
                out_hbm.at[pl.ds(base + j * _CHUNK, _CHUNK)],
                wsems.at[j % _NB],
            )
        ]

    gathers, writes = {}, {}
    waited = set()
    for j in range(min(2, _NCHUNKS)):
        gathers[j] = gather(j)
    for i in range(_NCHUNKS):
        gathers[i].wait()
        writes[i] = write(i)
        j = i + 2
        if j < _NCHUNKS:
            if j - _NB >= 0:
                for h in writes[j - _NB]:
                    h.wait()
                waited.add(j - _NB)
            gathers[j] = gather(j)
    for i in range(_NCHUNKS):
        if i not in waited:
            for h in writes[i]:
                h.wait()


def kernel(x, weight):
    wt = weight.T
    w128 = _pack_table(wt, wt, wt, wt)
    wlin = w128.reshape(_WSTRIDE * 4, _EMB)
    q = x // _WSTRIDE
    rows = (4 * x - (4 * _WSTRIDE - 1) * q).reshape(_N)
    out = _emb_lookup(rows, wlin)
    return out.reshape(_BATCH, _FIELDS, _EMB)


# speedup vs baseline: 2.6981x; 1.0004x over previous
import functools

import jax
import jax.numpy as jnp
from jax import lax
from jax.experimental import pallas as pl
from jax.experimental.pallas import tpu as pltpu
from jax.experimental.pallas import tpu_sc as plsc

_VOCAB = 1000000
_EMB = 32
_BATCH = 16384
_FIELDS = 26
_N = _BATCH * _FIELDS
_NW = 32
_B_PER_W = _N // _NW
_CHUNK = 1024
_NCHUNKS = _B_PER_W // _CHUNK
_NB = 3

_TCOLS = 8192
_TBLK = 31
_WSTRIDE = _TCOLS * _TBLK
_IN_BLKS = (_VOCAB + _TCOLS - 1) // _TCOLS - 1


def _transpose_body(x0_ref, x1_ref, x2_ref, x3_ref, o_ref):
    o_ref[...] = jnp.concatenate(
        [x0_ref[...].T, x1_ref[...].T, x2_ref[...].T, x3_ref[...].T], axis=1
    )


def _in_spec(q):
    return pl.BlockSpec(
        (32, _TCOLS), lambda j, q=q: (0, jnp.minimum(j + _TBLK * q, _IN_BLKS))
    )


_pack_table = pl.pallas_call(
    _transpose_body,
    grid=(_TBLK,),
    in_specs=[_in_spec(0), _in_spec(1), _in_spec(2), _in_spec(3)],
    out_specs=pl.BlockSpec((_TCOLS, 128), lambda j: (j, 0)),
    out_shape=jax.ShapeDtypeStruct((_WSTRIDE, 128), jnp.float32),
)

_mesh = plsc.VectorSubcoreMesh(core_axis_name="c", subcore_axis_name="s")


@functools.partial(
    pl.kernel,
    mesh=_mesh,
    out_type=jax.ShapeDtypeStruct((_N, _EMB), jnp.float32),
    scratch_types=[
        pltpu.VMEM((_B_PER_W,), jnp.int32),
        pltpu.VMEM((_NB, _CHUNK, _EMB), jnp.float32),
        pltpu.SemaphoreType.DMA((_NB,)),
        pltpu.SemaphoreType.DMA((_NB,)),
    ],
    compiler_params=pltpu.CompilerParams(use_tc_tiling_on_sc=False),
)
def _emb_lookup(idx_hbm, table_hbm, out_hbm, idx_v, rows_v, gsems, wsems):
    wid = lax.axis_index("s") * 2 + lax.axis_index("c")
    base = wid * _B_PER_W
    pltpu.sync_copy(idx_hbm.at[pl.ds(base, _B_PER_W)], idx_v)

    def gather(j):
        return pltpu.async_copy(
            table_hbm.at[idx_v.at[pl.ds(j * _CHUNK, _CHUNK)]],
            rows_v.at[j % _NB],
            gsems.at[j % _NB],
        )

    def write(j):
        return [
            pltpu.async_copy(
                rows_v.at[j % _NB],
                out_hbm.at[pl.ds(base + j * _CHUNK, _CHUNK)],
                wsems.at[j % _NB],
            )
        ]

    gathers, writes = {}, {}
    waited = set()
    for j in range(min(2, _NCHUNKS)):
        gathers[j] = gather(j)
    for i in range(_NCHUNKS):
        gathers[i].wait()
        writes[i] = write(i)
        j = i + 2
        if j < _NCHUNKS:
            if j - _NB >= 0:
                for h in writes[j - _NB]:
                    h.wait()
                waited.add(j - _NB)
            gathers[j] = gather(j)
    for i in range(_NCHUNKS):
        if i not in waited:
            for h in writes[i]:
                h.wait()


def kernel(x, weight):
    wt = weight.T
    w128 = _pack_table(wt, wt, wt, wt)
    wlin = w128.reshape(_WSTRIDE * 4, _EMB)
    q = x // _WSTRIDE
    rows = (4 * x - (4 * _WSTRIDE - 1) * q).reshape(_N)
    out = _emb_lookup(rows, wlin)
    return out.reshape(_BATCH, _FIELDS, _EMB)
